# baseline (speedup 1.0000x reference)
import functools

import jax
import jax.numpy as jnp
from jax import lax
from jax.experimental import pallas as pl
from jax.experimental.pallas import tpu as pltpu
from jax.experimental.pallas import tpu_sc as plsc

B, H, N, M = 8, 16, 2048, 2048
R = B + 1
IB = 128
BN = 2048
NW = 32
CHUNK = B * N // NW
L = 16


def _sum_body(imp_ref, t_ref, out_ref):
    out_ref[0:B, :] = jnp.sum(imp_ref[...], axis=1)
    out_ref[B:R, :] = t_ref[...]


_sum_call = pl.pallas_call(
    _sum_body,
    in_specs=[
        pl.BlockSpec((B, H, N), lambda: (0, 0, 0)),
        pl.BlockSpec((1, N), lambda: (0, 0)),
    ],
    out_specs=pl.BlockSpec((R, N), lambda: (0, 0)),
    out_shape=jax.ShapeDtypeStruct((R, N), jnp.float32),
)


def _rank_body(row_ref, sq_ref, rank_ref):
    row = row_ref[0]
    cols = jnp.transpose(sq_ref[0], (1, 0))
    i_l = lax.broadcasted_iota(jnp.int32, (IB, 1), 0)
    j_l = lax.broadcasted_iota(jnp.int32, (IB, IB), 1)
    tri = j_l < i_l
    NB = N // IB
    acc_col = [None] * NB
    acc_row = [None] * NB

    def acc(x, y):
        return y if x is None else x + y

    for a in range(NB):
        col = cols[:, a:a + 1]
        d = row[:, a * IB:(a + 1) * IB]
        dd = (d > col) | ((d == col) & tri)
        acc_col[a] = acc(acc_col[a], dd.astype(jnp.float32))
        for b in range(a + 1, NB):
            m = (row[:, b * IB:(b + 1) * IB] > col).astype(jnp.float32)
            acc_col[a] = acc_col[a] + m
            acc_row[b] = acc(acc_row[b], m)

    colcnt = jnp.concatenate(
        [jnp.sum(acc_col[a], axis=1, keepdims=True) for a in range(NB)],
        axis=1)
    colcnt_t = jnp.transpose(colcnt, (1, 0))
    rowparts = [jnp.zeros((1, IB), jnp.float32)] + [
        b * IB - jnp.sum(acc_row[b], axis=0, keepdims=True)
        for b in range(1, NB)]
    rowcnt = jnp.concatenate(rowparts, axis=0)
    rank_ref[0] = (colcnt_t + rowcnt).astype(jnp.int32)


NC = N // 128

_rank_call = pl.pallas_call(
    _rank_body,
    grid=(R,),
    in_specs=[
        pl.BlockSpec((1, 1, N), lambda b: (b, 0, 0)),
        pl.BlockSpec((1, NC, 128), lambda b: (b, 0, 0)),
    ],
    out_specs=pl.BlockSpec((1, NC, 128), lambda b: (b, 0, 0)),
    out_shape=jax.ShapeDtypeStruct((R, NC, 128), jnp.int32),
)


HB = 1024
HG = B * N // 2 // HB


def _maxarg_body(sa_ref, sb_ref, score_a, idx_a, score_b, idx_b):
    j = lax.broadcasted_iota(jnp.int32, (HB, M), 1)
    for x_ref, s_ref, i_ref in ((sa_ref, score_a, idx_a),
                                (sb_ref, score_b, idx_b)):
        x = x_ref[...]
        m = jnp.max(x, axis=-1, keepdims=True)
        idx = jnp.min(jnp.where(x == m, j, M), axis=-1, keepdims=True)
        s_ref[...] = m
        i_ref[...] = idx


_maxarg_call = pl.pallas_call(
    _maxarg_body,
    grid=(HG,),
    in_specs=[
        pl.BlockSpec((HB, M), lambda i: (i, 0)),
        pl.BlockSpec((HB, M), lambda i: (i + HG, 0)),
    ],
    out_specs=[
        pl.BlockSpec((HB, 1), lambda i: (i, 0)),
        pl.BlockSpec((HB, 1), lambda i: (i, 0)),
        pl.BlockSpec((HB, 1), lambda i: (i, 0)),
        pl.BlockSpec((HB, 1), lambda i: (i, 0)),
    ],
    out_shape=[
        jax.ShapeDtypeStruct((B * N // 2, 1), jnp.float32),
        jax.ShapeDtypeStruct((B * N // 2, 1), jnp.int32),
        jax.ShapeDtypeStruct((B * N // 2, 1), jnp.float32),
        jax.ShapeDtypeStruct((B * N // 2, 1), jnp.int32),
    ],
)


@functools.cache
def _make_sc_thr():
    mesh = plsc.VectorSubcoreMesh(core_axis_name="c", subcore_axis_name="s")

    @functools.partial(
        pl.kernel,
        mesh=mesh,
        compiler_params=pltpu.CompilerParams(needs_layout_passes=False),
        out_type=jax.ShapeDtypeStruct((B * N,), jnp.float32),
        scratch_types=[
            pltpu.VMEM((N,), jnp.int32),
            pltpu.VMEM((N,), jnp.float32),
            pltpu.VMEM((N,), jnp.float32),
            pltpu.VMEM((CHUNK,), jnp.int32),
            pltpu.VMEM((CHUNK,), jnp.float32),
            pltpu.VMEM((CHUNK,), jnp.float32),
            pltpu.SemaphoreType.DMA,
        ],
    )
    def _sc_thr(rank_hbm, t_hbm, cm_hbm, thr_out,
                rankt_v, t_v, tsort_v, rank_v, cm_v, thr_v, sem):
        wid = lax.axis_index("s") * 2 + lax.axis_index("c")
        base = wid * CHUNK
        c1 = pltpu.make_async_copy(rank_hbm.at[pl.ds(B * N, N)], rankt_v, sem)
        c2 = pltpu.make_async_copy(t_hbm, t_v, sem)
        c3 = pltpu.make_async_copy(rank_hbm.at[pl.ds(base, CHUNK)], rank_v, sem)
        c4 = pltpu.make_async_copy(cm_hbm.at[pl.ds(base, CHUNK)], cm_v, sem)
        c1.start(); c2.start(); c3.start(); c4.start()
        c1.wait(); c2.wait(); c3.wait(); c4.wait()

        def scat(k, c):
            s = k * L
            plsc.store_scatter(tsort_v, [rankt_v[pl.ds(s, L)]], t_v[pl.ds(s, L)])
            return c

        lax.fori_loop(0, N // L, scat, 0)

        def gath(k, c):
            s = k * L
            tv = plsc.load_gather(tsort_v, [rank_v[pl.ds(s, L)]])
            thr_v[pl.ds(s, L)] = tv - cm_v[pl.ds(s, L)]
            return c

        lax.fori_loop(0, CHUNK // L, gath, 0)

        pltpu.sync_copy(thr_v, thr_out.at[pl.ds(base, CHUNK)])

    return _sc_thr


def _sub_body(score_ref, thr_ref, out_ref):
    out_ref[...] = score_ref[...] - thr_ref[...]


_sub_call = pl.pallas_call(
    _sub_body,
    in_specs=[
        pl.BlockSpec((128, 128), lambda: (0, 0)),
        pl.BlockSpec((128, 128), lambda: (0, 0)),
    ],
    out_specs=pl.BlockSpec((128, 128), lambda: (0, 0)),
    out_shape=jax.ShapeDtypeStruct((128, 128), jnp.float32),
)


def kernel(importance, similarity, compressed_map, sim_threshold):
    vals = _sum_call(importance, sim_threshold[None, :])
    rank = _rank_call(vals.reshape(R, 1, N), vals.reshape(R, NC, 128))

    thr_flat = _make_sc_thr()(
        rank.reshape(R * N),
        sim_threshold,
        compressed_map.reshape(B * N),
    )
    sim2 = similarity.reshape(B * N, M)
    sa, ia, sb, ib2 = _maxarg_call(sim2, sim2)
    score2 = jnp.concatenate([sa, sb], axis=0)
    idx2 = jnp.concatenate([ia, ib2], axis=0)

    reuse = _sub_call(score2.reshape(128, 128), thr_flat.reshape(128, 128))
    return (
        reuse.reshape(B, N, 1),
        idx2.reshape(B, N),
        thr_flat.reshape(B, N),
    )


# speedup vs baseline: 1.0172x; 1.0172x over previous
import functools

import jax
import jax.numpy as jnp
from jax import lax
from jax.experimental import pallas as pl
from jax.experimental.pallas import tpu as pltpu
from jax.experimental.pallas import tpu_sc as plsc

B, H, N, M = 8, 16, 2048, 2048
R = B + 1
IB = 128
BN = 2048
NW = 32
CHUNK = B * N // NW
L = 16


def _sum_body(imp_ref, t_ref, out_ref):
    out_ref[0:B, :] = jnp.sum(imp_ref[...], axis=1)
    out_ref[B:R, :] = t_ref[...]


_sum_call = pl.pallas_call(
    _sum_body,
    in_specs=[
        pl.BlockSpec((B, H, N), lambda: (0, 0, 0)),
        pl.BlockSpec((1, N), lambda: (0, 0)),
    ],
    out_specs=pl.BlockSpec((R, N), lambda: (0, 0)),
    out_shape=jax.ShapeDtypeStruct((R, N), jnp.float32),
)


def _rank_body(row_ref, sq_ref, rank_ref):
    row = row_ref[0]
    cols = jnp.transpose(sq_ref[0], (1, 0))
    i_l = lax.broadcasted_iota(jnp.int32, (IB, 1), 0)
    j_l = lax.broadcasted_iota(jnp.int32, (IB, IB), 1)
    tri = j_l < i_l
    NB = N // IB
    acc_col = [None] * NB
    acc_row = [None] * NB

    def acc(x, y):
        return y if x is None else x + y

    for a in range(NB):
        col = cols[:, a:a + 1]
        d = row[:, a * IB:(a + 1) * IB]
        dd = (d > col) | ((d == col) & tri)
        acc_col[a] = acc(acc_col[a], dd.astype(jnp.float32))
        for b in range(a + 1, NB):
            m = (row[:, b * IB:(b + 1) * IB] > col).astype(jnp.float32)
            acc_col[a] = acc_col[a] + m
            acc_row[b] = acc(acc_row[b], m)

    colcnt = jnp.concatenate(
        [jnp.sum(acc_col[a], axis=1, keepdims=True) for a in range(NB)],
        axis=1)
    colcnt_t = jnp.transpose(colcnt, (1, 0))
    rowparts = [jnp.zeros((1, IB), jnp.float32)] + [
        b * IB - jnp.sum(acc_row[b], axis=0, keepdims=True)
        for b in range(1, NB)]
    rowcnt = jnp.concatenate(rowparts, axis=0)
    rank_ref[0] = (colcnt_t + rowcnt).astype(jnp.int32)


NC = N // 128

_rank_call = pl.pallas_call(
    _rank_body,
    grid=(R,),
    in_specs=[
        pl.BlockSpec((1, 1, N), lambda b: (b, 0, 0)),
        pl.BlockSpec((1, NC, 128), lambda b: (b, 0, 0)),
    ],
    out_specs=pl.BlockSpec((1, NC, 128), lambda b: (b, 0, 0)),
    out_shape=jax.ShapeDtypeStruct((R, NC, 128), jnp.int32),
)


def _maxarg_body(sim_ref, score_ref, idx_ref):
    x = sim_ref[...]
    m = jnp.max(x, axis=-1, keepdims=True)
    j = lax.broadcasted_iota(jnp.int32, (BN, M), 1)
    idx = jnp.min(jnp.where(x == m, j, M), axis=-1, keepdims=True)
    score_ref[...] = m
    idx_ref[...] = idx


_maxarg_call = pl.pallas_call(
    _maxarg_body,
    grid=(B * N // BN,),
    in_specs=[
        pl.BlockSpec((BN, M), lambda i: (i, 0)),
    ],
    out_specs=[
        pl.BlockSpec((BN, 1), lambda i: (i, 0)),
        pl.BlockSpec((BN, 1), lambda i: (i, 0)),
    ],
    out_shape=[
        jax.ShapeDtypeStruct((B * N, 1), jnp.float32),
        jax.ShapeDtypeStruct((B * N, 1), jnp.int32),
    ],
)


@functools.cache
def _make_sc_thr():
    mesh = plsc.VectorSubcoreMesh(core_axis_name="c", subcore_axis_name="s")

    @functools.partial(
        pl.kernel,
        mesh=mesh,
        compiler_params=pltpu.CompilerParams(needs_layout_passes=False),
        out_type=jax.ShapeDtypeStruct((B * N,), jnp.float32),
        scratch_types=[
            pltpu.VMEM((N,), jnp.int32),
            pltpu.VMEM((N,), jnp.float32),
            pltpu.VMEM((N,), jnp.float32),
            pltpu.VMEM((CHUNK,), jnp.int32),
            pltpu.VMEM((CHUNK,), jnp.float32),
            pltpu.VMEM((CHUNK,), jnp.float32),
            pltpu.SemaphoreType.DMA,
        ],
    )
    def _sc_thr(rank_hbm, t_hbm, cm_hbm, thr_out,
                rankt_v, t_v, tsort_v, rank_v, cm_v, thr_v, sem):
        wid = lax.axis_index("s") * 2 + lax.axis_index("c")
        base = wid * CHUNK
        c1 = pltpu.make_async_copy(rank_hbm.at[pl.ds(B * N, N)], rankt_v, sem)
        c2 = pltpu.make_async_copy(t_hbm, t_v, sem)
        c3 = pltpu.make_async_copy(rank_hbm.at[pl.ds(base, CHUNK)], rank_v, sem)
        c4 = pltpu.make_async_copy(cm_hbm.at[pl.ds(base, CHUNK)], cm_v, sem)
        c1.start(); c2.start(); c3.start(); c4.start()
        c1.wait(); c2.wait(); c3.wait(); c4.wait()

        def scat(k, c):
            s = k * L
            plsc.store_scatter(tsort_v, [rankt_v[pl.ds(s, L)]], t_v[pl.ds(s, L)])
            return c

        lax.fori_loop(0, N // L, scat, 0)

        def gath(k, c):
            s = k * L
            tv = plsc.load_gather(tsort_v, [rank_v[pl.ds(s, L)]])
            thr_v[pl.ds(s, L)] = tv - cm_v[pl.ds(s, L)]
            return c

        lax.fori_loop(0, CHUNK // L, gath, 0)

        pltpu.sync_copy(thr_v, thr_out.at[pl.ds(base, CHUNK)])

    return _sc_thr


def _sub_body(score_ref, thr_ref, out_ref):
    out_ref[...] = score_ref[...] - thr_ref[...]


_sub_call = pl.pallas_call(
    _sub_body,
    in_specs=[
        pl.BlockSpec((128, 128), lambda: (0, 0)),
        pl.BlockSpec((128, 128), lambda: (0, 0)),
    ],
    out_specs=pl.BlockSpec((128, 128), lambda: (0, 0)),
    out_shape=jax.ShapeDtypeStruct((128, 128), jnp.float32),
)


def kernel(importance, similarity, compressed_map, sim_threshold):
    vals = _sum_call(importance, sim_threshold[None, :])
    rank = _rank_call(vals.reshape(R, 1, N), vals.reshape(R, NC, 128))

    thr_flat = _make_sc_thr()(
        rank.reshape(R * N),
        sim_threshold,
        compressed_map.reshape(B * N),
    )
    score2, idx2 = _maxarg_call(similarity.reshape(B * N, M))

    reuse = _sub_call(score2.reshape(128, 128), thr_flat.reshape(128, 128))
    return (
        reuse.reshape(B, N, 1),
        idx2.reshape(B, N),
        thr_flat.reshape(B, N),
    )
